# baseline (speedup 1.0000x reference)
import functools
import jax
import jax.numpy as jnp
from jax import lax
from jax.experimental import pallas as pl
from jax.experimental.pallas import tpu as pltpu
from jax.experimental.pallas import tpu_sc as plsc

MAXLEN = 8192
DM = 1024
NB = 4
SL = 8192

_info = plsc.get_sparse_core_info()
_NC, _NS, _L = _info.num_cores, _info.num_subcores, _info.num_lanes
_NW = _NC * _NS
_ROWS_PER_W = SL // _NW
_CH = 32
_NCHUNK = _ROWS_PER_W // _CH
_CHW = _CH * DM
_NVEC = _CHW // _L


def _sc_body(x_hbm, t_hbm, o_hbm, xbuf, tbuf, obuf):
    wid = lax.axis_index("s") * _NC + lax.axis_index("c")
    row0 = wid * _ROWS_PER_W

    def do_chunk(ci, _):
        r = (row0 + ci * _CH) * DM
        pltpu.sync_copy(t_hbm.at[pl.ds(r, _CHW)], tbuf)

        def do_batch(b, _2):
            off = b * (SL * DM) + r
            pltpu.sync_copy(x_hbm.at[pl.ds(off, _CHW)], xbuf)

            def add_vec(i, _3):
                s = pl.ds(i * _L, _L)
                obuf[s] = xbuf[s] + tbuf[s]
                return 0

            lax.fori_loop(0, _NVEC, add_vec, 0, unroll=8)
            pltpu.sync_copy(obuf, o_hbm.at[pl.ds(off, _CHW)])
            return 0

        lax.fori_loop(0, NB, do_batch, 0)
        return 0

    lax.fori_loop(0, _NCHUNK, do_chunk, 0)


def kernel(x, embedding_table):
    B, S, D = x.shape
    xf = x.reshape(-1)
    tf = embedding_table.reshape(-1)
    run = pl.kernel(
        _sc_body,
        out_type=jax.ShapeDtypeStruct((B * S * D,), jnp.float32),
        mesh=plsc.VectorSubcoreMesh(core_axis_name="c", subcore_axis_name="s"),
        scratch_types=[
            pltpu.VMEM((_CHW,), jnp.float32),
            pltpu.VMEM((_CHW,), jnp.float32),
            pltpu.VMEM((_CHW,), jnp.float32),
        ],
    )
    out = run(xf, tf)
    return out.reshape(B, S, D)


# speedup vs baseline: 1.5516x; 1.5516x over previous
import functools
import jax
import jax.numpy as jnp
from jax import lax
from jax.experimental import pallas as pl
from jax.experimental.pallas import tpu as pltpu
from jax.experimental.pallas import tpu_sc as plsc

MAXLEN = 8192
DM = 1024
NB = 4
SL = 8192

_info = plsc.get_sparse_core_info()
_NC, _NS, _L = _info.num_cores, _info.num_subcores, _info.num_lanes
_NW = _NC * _NS
_ROWS_PER_W = SL // _NW
_CH = 32
_NCHUNK = _ROWS_PER_W // _CH
_CHW = _CH * DM
_NVEC = _CHW // _L


def _sc_body(x_hbm, t_hbm, o_hbm, xbuf, tbuf, obuf):
    wid = lax.axis_index("s") * _NC + lax.axis_index("c")
    row0 = wid * _ROWS_PER_W

    def do_chunk(ci, _):
        r = (row0 + ci * _CH) * DM
        pltpu.sync_copy(t_hbm.at[pl.ds(r, _CHW)], tbuf)

        def do_batch(b, _2):
            off = b * (SL * DM) + r
            pltpu.sync_copy(x_hbm.at[pl.ds(off, _CHW)], xbuf)

            @plsc.parallel_loop(0, _NVEC, unroll=8)
            def _add_vec(i):
                s = pl.ds(i * _L, _L)
                obuf[s] = xbuf[s] + tbuf[s]
            pltpu.sync_copy(obuf, o_hbm.at[pl.ds(off, _CHW)])
            return 0

        lax.fori_loop(0, NB, do_batch, 0)
        return 0

    lax.fori_loop(0, _NCHUNK, do_chunk, 0)


def kernel(x, embedding_table):
    B, S, D = x.shape
    xf = x.reshape(-1)
    tf = embedding_table.reshape(-1)
    run = pl.kernel(
        _sc_body,
        out_type=jax.ShapeDtypeStruct((B * S * D,), jnp.float32),
        mesh=plsc.VectorSubcoreMesh(core_axis_name="c", subcore_axis_name="s"),
        scratch_types=[
            pltpu.VMEM((_CHW,), jnp.float32),
            pltpu.VMEM((_CHW,), jnp.float32),
            pltpu.VMEM((_CHW,), jnp.float32),
        ],
    )
    out = run(xf, tf)
    return out.reshape(B, S, D)


# speedup vs baseline: 7.9914x; 5.1506x over previous
import jax
import jax.numpy as jnp
from jax.experimental import pallas as pl
from jax.experimental.pallas import tpu as pltpu


def _add_body(x_ref, t_ref, o_ref):
    o_ref[...] = x_ref[...] + t_ref[...]


def kernel(x, embedding_table):
    B, S, D = x.shape
    BS = 2048
    grid = (S // BS, B)
    return pl.pallas_call(
        _add_body,
        grid=grid,
        in_specs=[
            pl.BlockSpec((1, BS, D), lambda s, b: (b, s, 0)),
            pl.BlockSpec((BS, D), lambda s, b: (s, 0)),
        ],
        out_specs=pl.BlockSpec((1, BS, D), lambda s, b: (b, s, 0)),
        out_shape=jax.ShapeDtypeStruct(x.shape, x.dtype),
        compiler_params=pltpu.CompilerParams(
            dimension_semantics=("parallel", "parallel"),
        ),
    )(x, embedding_table)
